# baseline (speedup 1.0000x reference)
import functools
import jax
import jax.numpy as jnp
from jax.experimental import pallas as pl
from jax.experimental.pallas import tpu as pltpu

NUM_NODES = 50000
NODE_DIM = 3
EDGE_DIM = 4
N_BLOCKS = 2
FC_OUT = NUM_NODES * NODE_DIM

_DEC_TILE = 7680
_EDGE_TILE = 6400
_UPD_TILE = 5000


def _silu(x):
    return x * jax.nn.sigmoid(x)


def _head_body(p, wi1, bi1, wi2, bi2, wi3, bi3, wf1, bf1, h_out):
    z = _silu(jnp.dot(p[...], wi1[...], preferred_element_type=jnp.float32) + bi1[...])
    z = _silu(jnp.dot(z, wi2[...], preferred_element_type=jnp.float32) + bi2[...])
    z = jnp.dot(z, wi3[...], preferred_element_type=jnp.float32) + bi3[...]
    h_out[...] = _silu(jnp.dot(z, wf1[...], preferred_element_type=jnp.float32) + bf1[...])


def _head(p, wi1, bi1, wi2, bi2, wi3, bi3, wf1, bf1):
    return pl.pallas_call(
        _head_body,
        out_shape=jax.ShapeDtypeStruct((1, 256), jnp.float32),
    )(p, wi1, bi1, wi2, bi2, wi3, bi3, wf1, bf1)


def _dec_body(h, wf2, bf2, out):
    out[...] = jnp.dot(h[...], wf2[...], preferred_element_type=jnp.float32) + bf2[...]


def _decode(h, wf2, bf2):
    n_t = pl.cdiv(FC_OUT, _DEC_TILE)
    return pl.pallas_call(
        _dec_body,
        grid=(n_t,),
        in_specs=[
            pl.BlockSpec((1, 256), lambda j: (0, 0)),
            pl.BlockSpec((256, _DEC_TILE), lambda j: (0, j)),
            pl.BlockSpec((1, _DEC_TILE), lambda j: (0, j)),
        ],
        out_specs=pl.BlockSpec((1, _DEC_TILE), lambda j: (0, j)),
        out_shape=jax.ShapeDtypeStruct((1, FC_OUT), jnp.float32),
    )(h, wf2, bf2)


def _edge_body(gd, gs, ea, w1d, w1s, w1e, b1, w2, b2, out):
    m = jnp.dot(gd[...], w1d[...], preferred_element_type=jnp.float32)
    m += jnp.dot(gs[...], w1s[...], preferred_element_type=jnp.float32)
    m += jnp.dot(ea[...], w1e[...], preferred_element_type=jnp.float32)
    m = _silu(m + b1[...])
    out[...] = _silu(jnp.dot(m, w2[...], preferred_element_type=jnp.float32) + b2[...])


def _edge_mlp(gd, gs, ea, w1d, w1s, w1e, b1, w2, b2):
    n_e = gd.shape[0]
    n_t = n_e // _EDGE_TILE
    full = lambda r, c: pl.BlockSpec((r, c), lambda j: (0, 0))
    return pl.pallas_call(
        _edge_body,
        grid=(n_t,),
        in_specs=[
            pl.BlockSpec((_EDGE_TILE, 3), lambda j: (j, 0)),
            pl.BlockSpec((_EDGE_TILE, 3), lambda j: (j, 0)),
            pl.BlockSpec((_EDGE_TILE, EDGE_DIM), lambda j: (j, 0)),
            full(3, 64), full(3, 64), full(EDGE_DIM, 64), full(1, 64),
            full(64, 64), full(1, 64),
        ],
        out_specs=pl.BlockSpec((_EDGE_TILE, 64), lambda j: (j, 0)),
        out_shape=jax.ShapeDtypeStruct((n_e, 64), jnp.float32),
    )(gd, gs, ea, w1d, w1s, w1e, b1, w2, b2)


def _upd_body(x, aa, ab, wux, wua, wub, b1, w2, b2, out):
    u = jnp.dot(x[...], wux[...], preferred_element_type=jnp.float32)
    u += jnp.dot(aa[...], wua[...], preferred_element_type=jnp.float32)
    u += jnp.dot(ab[...], wub[...], preferred_element_type=jnp.float32)
    u = _silu(u + b1[...])
    u = jnp.dot(u, w2[...], preferred_element_type=jnp.float32) + b2[...]
    out[...] = x[...] + u


def _update(x, aggr_a, aggr_b, wux, wua, wub, b1, w2, b2):
    n_t = NUM_NODES // _UPD_TILE
    full = lambda r, c: pl.BlockSpec((r, c), lambda j: (0, 0))
    return pl.pallas_call(
        _upd_body,
        grid=(n_t,),
        in_specs=[
            pl.BlockSpec((_UPD_TILE, 3), lambda j: (j, 0)),
            pl.BlockSpec((_UPD_TILE, 32), lambda j: (j, 0)),
            pl.BlockSpec((_UPD_TILE, 32), lambda j: (j, 0)),
            full(3, 64), full(32, 64), full(32, 64), full(1, 64),
            full(64, 3), full(1, 3),
        ],
        out_specs=pl.BlockSpec((_UPD_TILE, 3), lambda j: (j, 0)),
        out_shape=jax.ShapeDtypeStruct((NUM_NODES, 3), jnp.float32),
    )(x, aggr_a, aggr_b, wux, wua, wub, b1, w2, b2)


def kernel(params, edge_index, edge_attr, batch_ids, weights):
    w = weights
    r1 = lambda b: b.reshape(1, -1)
    src = edge_index[0]
    dst = edge_index[1]

    h = _head(params, w['Wi1'], r1(w['bi1']), w['Wi2'], r1(w['bi2']),
              w['Wi3'], r1(w['bi3']), w['Wf1'], r1(w['bf1']))
    xf = _decode(h, w['Wf2'], r1(w['bf2']))
    x = xf.reshape(NUM_NODES, NODE_DIM)

    for l in range(N_BLOCKS):
        w1 = w['Wm1_%d' % l]
        gd = jnp.take(x, dst, axis=0)
        gs = jnp.take(x, src, axis=0)
        m2 = _edge_mlp(gd, gs, edge_attr,
                       w1[0:3], w1[3:6], w1[6:10], r1(w['bm1_%d' % l]),
                       w['Wm2_%d' % l], r1(w['bm2_%d' % l]))
        aggr = jax.ops.segment_sum(m2, dst, num_segments=NUM_NODES)
        wu1 = w['Wu1_%d' % l]
        x = _update(x, aggr[:, :32], aggr[:, 32:],
                    wu1[0:3], wu1[3:35], wu1[35:67], r1(w['bu1_%d' % l]),
                    w['Wu2_%d' % l], r1(w['bu2_%d' % l]))

    return x.reshape(1, NUM_NODES, NODE_DIM)


# speedup vs baseline: 1.7428x; 1.7428x over previous
import functools
import jax
import jax.numpy as jnp
from jax import lax
from jax.experimental import pallas as pl
from jax.experimental.pallas import tpu as pltpu
from jax.experimental.pallas import tpu_sc as plsc

NUM_NODES = 50000
NODE_DIM = 3
EDGE_DIM = 4
N_EDGES = 800000
N_BLOCKS = 2
FC_OUT = NUM_NODES * NODE_DIM

E_PAD = 819200

_DEC_TILE = 7680
_EDGE_TILE = 6400
_UPD_TILE = 5000

_G_PER_W = E_PAD // 32
_GSEG = 2560
_GCHUNK = 128

_S_ROWS_PER_W = (E_PAD // 128) // 16
_S_SEG_ROWS = 8
_S_SEG = _S_SEG_ROWS * 128
_TAB_ROWS_PER_W = NUM_NODES // 16


def _silu(x):
    return x * jax.nn.sigmoid(x)


def _head_body(p, wi1, bi1, wi2, bi2, wi3, bi3, wf1, bf1, h_out):
    z = _silu(jnp.dot(p[...], wi1[...], preferred_element_type=jnp.float32) + bi1[...])
    z = _silu(jnp.dot(z, wi2[...], preferred_element_type=jnp.float32) + bi2[...])
    z = jnp.dot(z, wi3[...], preferred_element_type=jnp.float32) + bi3[...]
    h_out[...] = _silu(jnp.dot(z, wf1[...], preferred_element_type=jnp.float32) + bf1[...])


def _head(p, wi1, bi1, wi2, bi2, wi3, bi3, wf1, bf1):
    return pl.pallas_call(
        _head_body,
        out_shape=jax.ShapeDtypeStruct((1, 256), jnp.float32),
    )(p, wi1, bi1, wi2, bi2, wi3, bi3, wf1, bf1)


def _dec_body(h, wf2, bf2, out):
    out[...] = jnp.dot(h[...], wf2[...], preferred_element_type=jnp.float32) + bf2[...]


def _decode(h, wf2, bf2):
    n_t = pl.cdiv(FC_OUT, _DEC_TILE)
    return pl.pallas_call(
        _dec_body,
        grid=(n_t,),
        in_specs=[
            pl.BlockSpec((1, 256), lambda j: (0, 0)),
            pl.BlockSpec((256, _DEC_TILE), lambda j: (0, j)),
            pl.BlockSpec((1, _DEC_TILE), lambda j: (0, j)),
        ],
        out_specs=pl.BlockSpec((1, _DEC_TILE), lambda j: (0, j)),
        out_shape=jax.ShapeDtypeStruct((1, FC_OUT), jnp.float32),
    )(h, wf2, bf2)


def _sc_gather(x4, dstp, srcp):
    mesh = plsc.VectorSubcoreMesh(core_axis_name="c", subcore_axis_name="s")

    @functools.partial(
        pl.kernel,
        mesh=mesh,
        out_type=[jax.ShapeDtypeStruct((E_PAD, 16), jnp.float32),
                  jax.ShapeDtypeStruct((E_PAD, 16), jnp.float32)],
        scratch_types=[pltpu.VMEM((_GSEG,), jnp.int32),
                       pltpu.VMEM((_GSEG, 16), jnp.float32),
                       pltpu.SemaphoreType.DMA],
        compiler_params=pltpu.CompilerParams(use_tc_tiling_on_sc=False),
    )
    def k(x_hbm, d_hbm, s_hbm, gd_hbm, gs_hbm, idx_v, rows_v, sem):
        wid = lax.axis_index("s") * 2 + lax.axis_index("c")
        base = wid * _G_PER_W

        def one(i_hbm, o_hbm):
            def seg(si, carry):
                off = pl.multiple_of(base + si * _GSEG, 8)
                pltpu.sync_copy(i_hbm.at[pl.ds(off, _GSEG)], idx_v)
                handles = []
                for ci in range(_GSEG // _GCHUNK):
                    co = ci * _GCHUNK
                    handles.append(pltpu.async_copy(
                        x_hbm.at[idx_v.at[pl.ds(co, _GCHUNK)]],
                        rows_v.at[pl.ds(co, _GCHUNK)], sem))
                for hd in handles:
                    hd.wait()
                pltpu.sync_copy(rows_v, o_hbm.at[pl.ds(off, _GSEG)])
                return carry

            lax.fori_loop(0, _G_PER_W // _GSEG, seg, 0)

        one(d_hbm, gd_hbm)
        one(s_hbm, gs_hbm)

    return k(x4, dstp, srcp)


def _sc_scatter(dst2d, m2q, zrows):
    mesh = plsc.VectorSubcoreMesh(core_axis_name="c", subcore_axis_name="s")

    @functools.partial(
        pl.kernel,
        mesh=mesh,
        out_type=[jax.ShapeDtypeStruct((NUM_NODES, 16), jnp.float32)] * 4,
        scratch_types=[pltpu.VMEM((_S_SEG_ROWS, 128), jnp.int32),
                       pltpu.VMEM((_S_SEG, 16), jnp.float32),
                       pltpu.VMEM_SHARED((NUM_NODES, 16), jnp.float32)],
        compiler_params=pltpu.CompilerParams(use_tc_tiling_on_sc=False),
    )
    def k(d_hbm, m0, m1, m2, m3, z_hbm, o0, o1, o2, o3, idx_v, vals_v, table):
        c = lax.axis_index("c")
        s = lax.axis_index("s")
        trow = pl.multiple_of(s * _TAB_ROWS_PER_W, 1)

        def one_pass(m_hbm, o_hbm):
            pltpu.sync_copy(z_hbm, table.at[pl.ds(trow, _TAB_ROWS_PER_W)])
            plsc.subcore_barrier()

            def seg(gi, carry):
                row0 = pl.multiple_of(s * _S_ROWS_PER_W + gi * _S_SEG_ROWS, 8)
                e0 = pl.multiple_of(row0 * 128, 8)
                pltpu.sync_copy(d_hbm.at[pl.ds(row0, _S_SEG_ROWS)], idx_v)
                pltpu.sync_copy(m_hbm.at[pl.ds(e0, _S_SEG)], vals_v)
                for j in range(_S_SEG_ROWS):
                    pltpu.sync_copy(vals_v.at[pl.ds(j * 128, 128)],
                                    table.at[idx_v.at[j]], add=True)
                return carry

            lax.fori_loop(0, _S_ROWS_PER_W // _S_SEG_ROWS, seg, 0)
            plsc.subcore_barrier()
            pltpu.sync_copy(table.at[pl.ds(trow, _TAB_ROWS_PER_W)],
                            o_hbm.at[pl.ds(trow, _TAB_ROWS_PER_W)])

        @pl.when(c == 0)
        def _():
            one_pass(m0, o0)
            one_pass(m1, o1)

        @pl.when(c == 1)
        def _():
            one_pass(m2, o2)
            one_pass(m3, o3)

    return k(dst2d, *m2q, zrows)


def _edge_body(gd, gs, ea, w1d, w1s, w1e, b1, w2, b2, o0, o1, o2, o3):
    m = jnp.dot(gd[...], w1d[...], preferred_element_type=jnp.float32)
    m += jnp.dot(gs[...], w1s[...], preferred_element_type=jnp.float32)
    m += jnp.dot(ea[...], w1e[...], preferred_element_type=jnp.float32)
    m = _silu(m + b1[...])
    m2 = _silu(jnp.dot(m, w2[...], preferred_element_type=jnp.float32) + b2[...])
    m2 = jnp.where(pl.program_id(0) < N_EDGES // _EDGE_TILE, m2, 0.0)
    o0[...] = m2[:, 0:16]
    o1[...] = m2[:, 16:32]
    o2[...] = m2[:, 32:48]
    o3[...] = m2[:, 48:64]


def _edge_mlp(gd, gs, ea, w1d, w1s, w1e, b1, w2, b2):
    n_t = E_PAD // _EDGE_TILE
    full = lambda r, c: pl.BlockSpec((r, c), lambda j: (0, 0))
    return pl.pallas_call(
        _edge_body,
        grid=(n_t,),
        in_specs=[
            pl.BlockSpec((_EDGE_TILE, 16), lambda j: (j, 0)),
            pl.BlockSpec((_EDGE_TILE, 16), lambda j: (j, 0)),
            pl.BlockSpec((_EDGE_TILE, EDGE_DIM), lambda j: (j, 0)),
            full(16, 64), full(16, 64), full(EDGE_DIM, 64), full(1, 64),
            full(64, 64), full(1, 64),
        ],
        out_specs=[pl.BlockSpec((_EDGE_TILE, 16), lambda j: (j, 0))] * 4,
        out_shape=[jax.ShapeDtypeStruct((E_PAD, 16), jnp.float32)] * 4,
    )(gd, gs, ea, w1d, w1s, w1e, b1, w2, b2)


def _upd_body(x, a0, a1, a2, a3, wux, wa0, wa1, wa2, wa3, b1, w2, b2, out):
    u = jnp.dot(x[...], wux[...], preferred_element_type=jnp.float32)
    u += jnp.dot(a0[...], wa0[...], preferred_element_type=jnp.float32)
    u += jnp.dot(a1[...], wa1[...], preferred_element_type=jnp.float32)
    u += jnp.dot(a2[...], wa2[...], preferred_element_type=jnp.float32)
    u += jnp.dot(a3[...], wa3[...], preferred_element_type=jnp.float32)
    u = _silu(u + b1[...])
    u = jnp.dot(u, w2[...], preferred_element_type=jnp.float32) + b2[...]
    out[...] = x[...] + u


def _update(x, aggr, wux, was, b1, w2, b2):
    n_t = NUM_NODES // _UPD_TILE
    full = lambda r, c: pl.BlockSpec((r, c), lambda j: (0, 0))
    return pl.pallas_call(
        _upd_body,
        grid=(n_t,),
        in_specs=[
            pl.BlockSpec((_UPD_TILE, 3), lambda j: (j, 0)),
            pl.BlockSpec((_UPD_TILE, 16), lambda j: (j, 0)),
            pl.BlockSpec((_UPD_TILE, 16), lambda j: (j, 0)),
            pl.BlockSpec((_UPD_TILE, 16), lambda j: (j, 0)),
            pl.BlockSpec((_UPD_TILE, 16), lambda j: (j, 0)),
            full(3, 64), full(16, 64), full(16, 64), full(16, 64),
            full(16, 64), full(1, 64), full(64, 3), full(1, 3),
        ],
        out_specs=pl.BlockSpec((_UPD_TILE, 3), lambda j: (j, 0)),
        out_shape=jax.ShapeDtypeStruct((NUM_NODES, 3), jnp.float32),
    )(x, *aggr, wux, *was, b1, w2, b2)


def kernel(params, edge_index, edge_attr, batch_ids, weights):
    w = weights
    r1 = lambda b: b.reshape(1, -1)
    pad_e = E_PAD - N_EDGES
    dstp = jnp.concatenate([edge_index[1], jnp.zeros((pad_e,), edge_index.dtype)])
    srcp = jnp.concatenate([edge_index[0], jnp.zeros((pad_e,), edge_index.dtype)])
    dst2d = dstp.reshape(E_PAD // 128, 128)
    eap = jnp.pad(edge_attr, ((0, pad_e), (0, 0)))
    zrows = jnp.zeros((_TAB_ROWS_PER_W, 16), jnp.float32)

    h = _head(params, w['Wi1'], r1(w['bi1']), w['Wi2'], r1(w['bi2']),
              w['Wi3'], r1(w['bi3']), w['Wf1'], r1(w['bf1']))
    xf = _decode(h, w['Wf2'], r1(w['bf2']))
    x = xf.reshape(NUM_NODES, NODE_DIM)

    pad16 = lambda m: jnp.pad(m, ((0, 13), (0, 0)))
    for l in range(N_BLOCKS):
        w1 = w['Wm1_%d' % l]
        x16 = jnp.pad(x, ((0, 0), (0, 13)))
        gd, gs = _sc_gather(x16, dstp, srcp)
        m2q = _edge_mlp(gd, gs, eap,
                        pad16(w1[0:3]), pad16(w1[3:6]), w1[6:10],
                        r1(w['bm1_%d' % l]),
                        w['Wm2_%d' % l], r1(w['bm2_%d' % l]))
        aggr = _sc_scatter(dst2d, m2q, zrows)
        wu1 = w['Wu1_%d' % l]
        x = _update(x, aggr,
                    wu1[0:3], [wu1[3 + 16 * q:3 + 16 * (q + 1)] for q in range(4)],
                    r1(w['bu1_%d' % l]),
                    w['Wu2_%d' % l], r1(w['bu2_%d' % l]))

    return x.reshape(1, NUM_NODES, NODE_DIM)


# speedup vs baseline: 2.0070x; 1.1516x over previous
import functools
import jax
import jax.numpy as jnp
from jax import lax
from jax.experimental import pallas as pl
from jax.experimental.pallas import tpu as pltpu
from jax.experimental.pallas import tpu_sc as plsc

NUM_NODES = 50000
NODE_DIM = 3
EDGE_DIM = 4
N_EDGES = 800000
N_BLOCKS = 2
FC_OUT = NUM_NODES * NODE_DIM

E_PAD = 819200

_DEC_TILE = 7680
_EDGE_TILE = 6400
_UPD_TILE = 5000

_G_PER_W = E_PAD // 32
_GSEG = 2560
_GCHUNK = 128

_S_ROWS_PER_W = (E_PAD // 128) // 16
_S_SEG_ROWS = 8
_S_SEG = _S_SEG_ROWS * 128
_TAB_ROWS_PER_W = NUM_NODES // 16


def _silu(x):
    return x * jax.nn.sigmoid(x)


def _head_body(p, wi1, bi1, wi2, bi2, wi3, bi3, wf1, bf1, h_out):
    z = _silu(jnp.dot(p[...], wi1[...], preferred_element_type=jnp.float32) + bi1[...])
    z = _silu(jnp.dot(z, wi2[...], preferred_element_type=jnp.float32) + bi2[...])
    z = jnp.dot(z, wi3[...], preferred_element_type=jnp.float32) + bi3[...]
    h_out[...] = _silu(jnp.dot(z, wf1[...], preferred_element_type=jnp.float32) + bf1[...])


def _head(p, wi1, bi1, wi2, bi2, wi3, bi3, wf1, bf1):
    return pl.pallas_call(
        _head_body,
        out_shape=jax.ShapeDtypeStruct((1, 256), jnp.float32),
    )(p, wi1, bi1, wi2, bi2, wi3, bi3, wf1, bf1)


def _dec_body(h, wf2, bf2, out):
    out[...] = jnp.dot(h[...], wf2[...], preferred_element_type=jnp.float32) + bf2[...]


def _decode(h, wf2, bf2):
    n_t = pl.cdiv(FC_OUT, _DEC_TILE)
    return pl.pallas_call(
        _dec_body,
        grid=(n_t,),
        in_specs=[
            pl.BlockSpec((1, 256), lambda j: (0, 0)),
            pl.BlockSpec((256, _DEC_TILE), lambda j: (0, j)),
            pl.BlockSpec((1, _DEC_TILE), lambda j: (0, j)),
        ],
        out_specs=pl.BlockSpec((1, _DEC_TILE), lambda j: (0, j)),
        out_shape=jax.ShapeDtypeStruct((1, FC_OUT), jnp.float32),
    )(h, wf2, bf2)


def _sc_gather(x4, dstp, srcp):
    mesh = plsc.VectorSubcoreMesh(core_axis_name="c", subcore_axis_name="s")

    @functools.partial(
        pl.kernel,
        mesh=mesh,
        out_type=[jax.ShapeDtypeStruct((E_PAD, 16), jnp.float32),
                  jax.ShapeDtypeStruct((E_PAD, 16), jnp.float32)],
        scratch_types=[pltpu.VMEM((_GSEG,), jnp.int32),
                       pltpu.VMEM((_GSEG, 16), jnp.float32),
                       pltpu.SemaphoreType.DMA],
        compiler_params=pltpu.CompilerParams(use_tc_tiling_on_sc=False),
    )
    def k(x_hbm, d_hbm, s_hbm, gd_hbm, gs_hbm, idx_v, rows_v, sem):
        wid = lax.axis_index("s") * 2 + lax.axis_index("c")
        base = wid * _G_PER_W

        def one(i_hbm, o_hbm):
            def seg(si, carry):
                off = pl.multiple_of(base + si * _GSEG, 8)
                pltpu.sync_copy(i_hbm.at[pl.ds(off, _GSEG)], idx_v)
                handles = []
                for ci in range(_GSEG // _GCHUNK):
                    co = ci * _GCHUNK
                    handles.append(pltpu.async_copy(
                        x_hbm.at[idx_v.at[pl.ds(co, _GCHUNK)]],
                        rows_v.at[pl.ds(co, _GCHUNK)], sem))
                for hd in handles:
                    hd.wait()
                pltpu.sync_copy(rows_v, o_hbm.at[pl.ds(off, _GSEG)])
                return carry

            lax.fori_loop(0, _G_PER_W // _GSEG, seg, 0)

        one(d_hbm, gd_hbm)
        one(s_hbm, gs_hbm)

    return k(x4, dstp, srcp)


def _sc_scatter(dst2d, m2q, zrows):
    mesh = plsc.VectorSubcoreMesh(core_axis_name="c", subcore_axis_name="s")

    @functools.partial(
        pl.kernel,
        mesh=mesh,
        out_type=[jax.ShapeDtypeStruct((NUM_NODES, 16), jnp.float32)] * 4,
        scratch_types=[pltpu.VMEM((_S_SEG_ROWS, 128), jnp.int32),
                       pltpu.VMEM((_S_SEG, 16), jnp.float32),
                       pltpu.VMEM_SHARED((NUM_NODES, 16), jnp.float32)],
        compiler_params=pltpu.CompilerParams(use_tc_tiling_on_sc=False),
    )
    def k(d_hbm, m0, m1, m2, m3, z_hbm, o0, o1, o2, o3, idx_v, vals_v, table):
        c = lax.axis_index("c")
        s = lax.axis_index("s")
        trow = pl.multiple_of(s * _TAB_ROWS_PER_W, 1)

        def one_pass(m_hbm, o_hbm):
            pltpu.sync_copy(z_hbm, table.at[pl.ds(trow, _TAB_ROWS_PER_W)])
            plsc.subcore_barrier()

            def seg(gi, carry):
                row0 = pl.multiple_of(s * _S_ROWS_PER_W + gi * _S_SEG_ROWS, 8)
                e0 = pl.multiple_of(row0 * 128, 8)
                pltpu.sync_copy(d_hbm.at[pl.ds(row0, _S_SEG_ROWS)], idx_v)
                pltpu.sync_copy(m_hbm.at[pl.ds(e0, _S_SEG)], vals_v)
                for j in range(_S_SEG_ROWS):
                    pltpu.sync_copy(vals_v.at[pl.ds(j * 128, 128)],
                                    table.at[idx_v.at[j]], add=True)
                return carry

            lax.fori_loop(0, _S_ROWS_PER_W // _S_SEG_ROWS, seg, 0)
            plsc.subcore_barrier()
            pltpu.sync_copy(table.at[pl.ds(trow, _TAB_ROWS_PER_W)],
                            o_hbm.at[pl.ds(trow, _TAB_ROWS_PER_W)])

        @pl.when(c == 0)
        def _():
            one_pass(m0, o0)
            one_pass(m1, o1)

        @pl.when(c == 1)
        def _():
            one_pass(m2, o2)
            one_pass(m3, o3)

    return k(dst2d, *m2q, zrows)


def _edge_body(gd, gs, ea, w1d, w1s, w1e, b1, w2, b2, o0, o1, o2, o3):
    m = jnp.dot(gd[...], w1d[...], preferred_element_type=jnp.float32)
    m += jnp.dot(gs[...], w1s[...], preferred_element_type=jnp.float32)
    m += lax.dot_general(ea[...], w1e[...], (((0,), (0,)), ((), ())),
                         preferred_element_type=jnp.float32)
    m = _silu(m + b1[...])
    m2 = _silu(jnp.dot(m, w2[...], preferred_element_type=jnp.float32) + b2[...])
    m2 = jnp.where(pl.program_id(0) < N_EDGES // _EDGE_TILE, m2, 0.0)
    o0[...] = m2[:, 0:16]
    o1[...] = m2[:, 16:32]
    o2[...] = m2[:, 32:48]
    o3[...] = m2[:, 48:64]


def _edge_mlp(gd, gs, ea, w1d, w1s, w1e, b1, w2, b2):
    n_t = E_PAD // _EDGE_TILE
    full = lambda r, c: pl.BlockSpec((r, c), lambda j: (0, 0))
    return pl.pallas_call(
        _edge_body,
        grid=(n_t,),
        in_specs=[
            pl.BlockSpec((_EDGE_TILE, 16), lambda j: (j, 0)),
            pl.BlockSpec((_EDGE_TILE, 16), lambda j: (j, 0)),
            pl.BlockSpec((EDGE_DIM, _EDGE_TILE), lambda j: (0, j)),
            full(16, 64), full(16, 64), full(EDGE_DIM, 64), full(1, 64),
            full(64, 64), full(1, 64),
        ],
        out_specs=[pl.BlockSpec((_EDGE_TILE, 16), lambda j: (j, 0))] * 4,
        out_shape=[jax.ShapeDtypeStruct((E_PAD, 16), jnp.float32)] * 4,
    )(gd, gs, ea, w1d, w1s, w1e, b1, w2, b2)


def _upd_body(x, a0, a1, a2, a3, wux, wa0, wa1, wa2, wa3, b1, w2, b2, out):
    u = jnp.dot(x[...], wux[...], preferred_element_type=jnp.float32)
    u += jnp.dot(a0[...], wa0[...], preferred_element_type=jnp.float32)
    u += jnp.dot(a1[...], wa1[...], preferred_element_type=jnp.float32)
    u += jnp.dot(a2[...], wa2[...], preferred_element_type=jnp.float32)
    u += jnp.dot(a3[...], wa3[...], preferred_element_type=jnp.float32)
    u = _silu(u + b1[...])
    u = jnp.dot(u, w2[...], preferred_element_type=jnp.float32) + b2[...]
    out[...] = x[...] + u


def _update(x, aggr, wux, was, b1, w2, b2):
    n_t = NUM_NODES // _UPD_TILE
    full = lambda r, c: pl.BlockSpec((r, c), lambda j: (0, 0))
    return pl.pallas_call(
        _upd_body,
        grid=(n_t,),
        in_specs=[
            pl.BlockSpec((_UPD_TILE, 3), lambda j: (j, 0)),
            pl.BlockSpec((_UPD_TILE, 16), lambda j: (j, 0)),
            pl.BlockSpec((_UPD_TILE, 16), lambda j: (j, 0)),
            pl.BlockSpec((_UPD_TILE, 16), lambda j: (j, 0)),
            pl.BlockSpec((_UPD_TILE, 16), lambda j: (j, 0)),
            full(3, 64), full(16, 64), full(16, 64), full(16, 64),
            full(16, 64), full(1, 64), full(64, 3), full(1, 3),
        ],
        out_specs=pl.BlockSpec((_UPD_TILE, 3), lambda j: (j, 0)),
        out_shape=jax.ShapeDtypeStruct((NUM_NODES, 3), jnp.float32),
    )(x, *aggr, wux, *was, b1, w2, b2)


def kernel(params, edge_index, edge_attr, batch_ids, weights):
    w = weights
    r1 = lambda b: b.reshape(1, -1)
    pad_e = E_PAD - N_EDGES
    dstp = jnp.concatenate([edge_index[1], jnp.zeros((pad_e,), edge_index.dtype)])
    srcp = jnp.concatenate([edge_index[0], jnp.zeros((pad_e,), edge_index.dtype)])
    dst2d = dstp.reshape(E_PAD // 128, 128)
    eapT = jnp.pad(edge_attr.T, ((0, 0), (0, pad_e)))
    zrows = jnp.zeros((_TAB_ROWS_PER_W, 16), jnp.float32)

    h = _head(params, w['Wi1'], r1(w['bi1']), w['Wi2'], r1(w['bi2']),
              w['Wi3'], r1(w['bi3']), w['Wf1'], r1(w['bf1']))
    xf = _decode(h, w['Wf2'], r1(w['bf2']))
    x = xf.reshape(NUM_NODES, NODE_DIM)

    pad16 = lambda m: jnp.pad(m, ((0, 13), (0, 0)))
    for l in range(N_BLOCKS):
        w1 = w['Wm1_%d' % l]
        x16 = jnp.pad(x, ((0, 0), (0, 13)))
        gd, gs = _sc_gather(x16, dstp, srcp)
        m2q = _edge_mlp(gd, gs, eapT,
                        pad16(w1[0:3]), pad16(w1[3:6]), w1[6:10],
                        r1(w['bm1_%d' % l]),
                        w['Wm2_%d' % l], r1(w['bm2_%d' % l]))
        aggr = _sc_scatter(dst2d, m2q, zrows)
        wu1 = w['Wu1_%d' % l]
        x = _update(x, aggr,
                    wu1[0:3], [wu1[3 + 16 * q:3 + 16 * (q + 1)] for q in range(4)],
                    r1(w['bu1_%d' % l]),
                    w['Wu2_%d' % l], r1(w['bu2_%d' % l]))

    return x.reshape(1, NUM_NODES, NODE_DIM)


# speedup vs baseline: 4.3375x; 2.1612x over previous
import functools
import jax
import jax.numpy as jnp
from jax import lax
from jax.experimental import pallas as pl
from jax.experimental.pallas import tpu as pltpu
from jax.experimental.pallas import tpu_sc as plsc

NUM_NODES = 50000
NODE_DIM = 3
EDGE_DIM = 4
N_EDGES = 800000
N_BLOCKS = 2
FC_OUT = NUM_NODES * NODE_DIM

E_PAD = 819200

_DEC_TILE = 7680
_EDGE_TILE = 8192
_UPD_TILE = 5000

_G_PER_W = E_PAD // 32
_GSEG = 2560
_GCHUNK = 128

_S_ROWS_PER_W = (E_PAD // 128) // 16
_S_SEG_ROWS = 8
_S_SEG = _S_SEG_ROWS * 128
_TAB_ROWS_PER_W = NUM_NODES // 16


def _silu(x):
    return x * jax.nn.sigmoid(x)


def _head_body(p, wi1, bi1, wi2, bi2, wi3, bi3, wf1, bf1, h_out):
    z = _silu(jnp.dot(p[...], wi1[...], preferred_element_type=jnp.float32) + bi1[...])
    z = _silu(jnp.dot(z, wi2[...], preferred_element_type=jnp.float32) + bi2[...])
    z = jnp.dot(z, wi3[...], preferred_element_type=jnp.float32) + bi3[...]
    h_out[...] = _silu(jnp.dot(z, wf1[...], preferred_element_type=jnp.float32) + bf1[...])


def _head(p, wi1, bi1, wi2, bi2, wi3, bi3, wf1, bf1):
    return pl.pallas_call(
        _head_body,
        out_shape=jax.ShapeDtypeStruct((1, 256), jnp.float32),
    )(p, wi1, bi1, wi2, bi2, wi3, bi3, wf1, bf1)


def _dec_body(h, wf2, bf2, out):
    out[...] = jnp.dot(h[...], wf2[...], preferred_element_type=jnp.float32) + bf2[...]


def _decode(h, wf2, bf2):
    n_t = pl.cdiv(FC_OUT, _DEC_TILE)
    return pl.pallas_call(
        _dec_body,
        grid=(n_t,),
        in_specs=[
            pl.BlockSpec((1, 256), lambda j: (0, 0)),
            pl.BlockSpec((256, _DEC_TILE), lambda j: (0, j)),
            pl.BlockSpec((1, _DEC_TILE), lambda j: (0, j)),
        ],
        out_specs=pl.BlockSpec((1, _DEC_TILE), lambda j: (0, j)),
        out_shape=jax.ShapeDtypeStruct((1, FC_OUT), jnp.float32),
    )(h, wf2, bf2)


def _sc_gather(x4, dstp, srcp):
    mesh = plsc.VectorSubcoreMesh(core_axis_name="c", subcore_axis_name="s")

    @functools.partial(
        pl.kernel,
        mesh=mesh,
        out_type=[jax.ShapeDtypeStruct((E_PAD, 16), jnp.float32),
                  jax.ShapeDtypeStruct((E_PAD, 16), jnp.float32)],
        scratch_types=[pltpu.VMEM((_GSEG,), jnp.int32),
                       pltpu.VMEM((_GSEG, 16), jnp.float32),
                       pltpu.SemaphoreType.DMA],
        compiler_params=pltpu.CompilerParams(use_tc_tiling_on_sc=False),
    )
    def k(x_hbm, d_hbm, s_hbm, gd_hbm, gs_hbm, idx_v, rows_v, sem):
        wid = lax.axis_index("s") * 2 + lax.axis_index("c")
        base = wid * _G_PER_W

        def one(i_hbm, o_hbm):
            def seg(si, carry):
                off = pl.multiple_of(base + si * _GSEG, 8)
                pltpu.sync_copy(i_hbm.at[pl.ds(off, _GSEG)], idx_v)
                handles = []
                for ci in range(_GSEG // _GCHUNK):
                    co = ci * _GCHUNK
                    handles.append(pltpu.async_copy(
                        x_hbm.at[idx_v.at[pl.ds(co, _GCHUNK)]],
                        rows_v.at[pl.ds(co, _GCHUNK)], sem))
                for hd in handles:
                    hd.wait()
                pltpu.sync_copy(rows_v, o_hbm.at[pl.ds(off, _GSEG)])
                return carry

            lax.fori_loop(0, _G_PER_W // _GSEG, seg, 0)

        one(d_hbm, gd_hbm)
        one(s_hbm, gs_hbm)

    return k(x4, dstp, srcp)


def _sc_scatter(dst2d, m2q, zrows):
    mesh = plsc.VectorSubcoreMesh(core_axis_name="c", subcore_axis_name="s")

    @functools.partial(
        pl.kernel,
        mesh=mesh,
        out_type=[jax.ShapeDtypeStruct((NUM_NODES, 16), jnp.float32)] * 4,
        scratch_types=[pltpu.VMEM((_S_SEG_ROWS, 128), jnp.int32),
                       pltpu.VMEM((_S_SEG, 16), jnp.float32),
                       pltpu.VMEM_SHARED((NUM_NODES, 16), jnp.float32)],
        compiler_params=pltpu.CompilerParams(use_tc_tiling_on_sc=False),
    )
    def k(d_hbm, m0, m1, m2, m3, z_hbm, o0, o1, o2, o3, idx_v, vals_v, table):
        c = lax.axis_index("c")
        s = lax.axis_index("s")
        trow = pl.multiple_of(s * _TAB_ROWS_PER_W, 1)

        def one_pass(m_hbm, o_hbm):
            pltpu.sync_copy(z_hbm, table.at[pl.ds(trow, _TAB_ROWS_PER_W)])
            plsc.subcore_barrier()

            def seg(gi, carry):
                row0 = pl.multiple_of(s * _S_ROWS_PER_W + gi * _S_SEG_ROWS, 8)
                e0 = pl.multiple_of(row0 * 128, 8)
                pltpu.sync_copy(d_hbm.at[pl.ds(row0, _S_SEG_ROWS)], idx_v)
                pltpu.sync_copy(m_hbm.at[pl.ds(e0, _S_SEG)], vals_v)
                for j in range(_S_SEG_ROWS):
                    pltpu.sync_copy(vals_v.at[pl.ds(j * 128, 128)],
                                    table.at[idx_v.at[j]], add=True)
                return carry

            lax.fori_loop(0, _S_ROWS_PER_W // _S_SEG_ROWS, seg, 0)
            plsc.subcore_barrier()
            pltpu.sync_copy(table.at[pl.ds(trow, _TAB_ROWS_PER_W)],
                            o_hbm.at[pl.ds(trow, _TAB_ROWS_PER_W)])

        @pl.when(c == 0)
        def _():
            one_pass(m0, o0)
            one_pass(m1, o1)

        @pl.when(c == 1)
        def _():
            one_pass(m2, o2)
            one_pass(m3, o3)

    return k(dst2d, *m2q, zrows)


def _edge_body(gd, gs, ea, wbd, wbs, wbe, b1p, w2bd, b2p, o0, o1, o2, o3):
    m = jnp.dot(gd[...], wbd[...], preferred_element_type=jnp.float32)
    m += jnp.dot(gs[...], wbs[...], preferred_element_type=jnp.float32)
    m += lax.dot_general(ea[...], wbe[...], (((0,), (0,)), ((), ())),
                         preferred_element_type=jnp.float32)
    m = _silu(m + b1p[...])
    m2 = _silu(jnp.dot(m, w2bd[...], preferred_element_type=jnp.float32) + b2p[...])
    row0 = pl.program_id(0) * (_EDGE_TILE // 8)
    rid = row0 + lax.broadcasted_iota(jnp.int32, (_EDGE_TILE // 8, 1), 0)
    m2 = jnp.where(rid < N_EDGES // 8, m2, 0.0)
    for q, o in enumerate((o0, o1, o2, o3)):
        o[...] = jnp.concatenate(
            [m2[:, 64 * j + 16 * q:64 * j + 16 * q + 16] for j in range(8)],
            axis=1)


def _edge_mlp(gd, gs, eA, wbd, wbs, wbe, b1p, w2bd, b2p):
    n_t = E_PAD // _EDGE_TILE
    rp = _EDGE_TILE // 8
    full = lambda r, c: pl.BlockSpec((r, c), lambda j: (0, 0))
    return pl.pallas_call(
        _edge_body,
        grid=(n_t,),
        in_specs=[
            pl.BlockSpec((rp, 128), lambda j: (j, 0)),
            pl.BlockSpec((rp, 128), lambda j: (j, 0)),
            pl.BlockSpec((32, rp), lambda j: (0, j)),
            full(128, 512), full(128, 512), full(32, 512), full(1, 512),
            full(512, 512), full(1, 512),
        ],
        out_specs=[pl.BlockSpec((rp, 128), lambda j: (j, 0))] * 4,
        out_shape=[jax.ShapeDtypeStruct((E_PAD // 8, 128), jnp.float32)] * 4,
    )(gd.reshape(E_PAD // 8, 128), gs.reshape(E_PAD // 8, 128),
      eA, wbd, wbs, wbe, b1p, w2bd, b2p)


def _upd_body(x, a0, a1, a2, a3, wux, wa0, wa1, wa2, wa3, b1, w2, b2, out):
    u = jnp.dot(x[...], wux[...], preferred_element_type=jnp.float32)
    u += jnp.dot(a0[...], wa0[...], preferred_element_type=jnp.float32)
    u += jnp.dot(a1[...], wa1[...], preferred_element_type=jnp.float32)
    u += jnp.dot(a2[...], wa2[...], preferred_element_type=jnp.float32)
    u += jnp.dot(a3[...], wa3[...], preferred_element_type=jnp.float32)
    u = _silu(u + b1[...])
    u = jnp.dot(u, w2[...], preferred_element_type=jnp.float32) + b2[...]
    out[...] = x[...] + u


def _update(x, aggr, wux, was, b1, w2, b2):
    n_t = NUM_NODES // _UPD_TILE
    full = lambda r, c: pl.BlockSpec((r, c), lambda j: (0, 0))
    return pl.pallas_call(
        _upd_body,
        grid=(n_t,),
        in_specs=[
            pl.BlockSpec((_UPD_TILE, 3), lambda j: (j, 0)),
            pl.BlockSpec((_UPD_TILE, 16), lambda j: (j, 0)),
            pl.BlockSpec((_UPD_TILE, 16), lambda j: (j, 0)),
            pl.BlockSpec((_UPD_TILE, 16), lambda j: (j, 0)),
            pl.BlockSpec((_UPD_TILE, 16), lambda j: (j, 0)),
            full(3, 64), full(16, 64), full(16, 64), full(16, 64),
            full(16, 64), full(1, 64), full(64, 3), full(1, 3),
        ],
        out_specs=pl.BlockSpec((_UPD_TILE, 3), lambda j: (j, 0)),
        out_shape=jax.ShapeDtypeStruct((NUM_NODES, 3), jnp.float32),
    )(x, *aggr, wux, *was, b1, w2, b2)


def kernel(params, edge_index, edge_attr, batch_ids, weights):
    w = weights
    r1 = lambda b: b.reshape(1, -1)
    pad_e = E_PAD - N_EDGES
    dstp = jnp.concatenate([edge_index[1], jnp.zeros((pad_e,), edge_index.dtype)])
    srcp = jnp.concatenate([edge_index[0], jnp.zeros((pad_e,), edge_index.dtype)])
    dst2d = dstp.reshape(E_PAD // 128, 128)
    eapT = jnp.pad(edge_attr.T, ((0, 0), (0, pad_e)))
    eA = eapT.reshape(4, E_PAD // 8, 8).transpose(0, 2, 1).reshape(32, E_PAD // 8)
    eye8 = jnp.eye(8, dtype=jnp.float32)
    zrows = jnp.zeros((_TAB_ROWS_PER_W, 16), jnp.float32)

    h = _head(params, w['Wi1'], r1(w['bi1']), w['Wi2'], r1(w['bi2']),
              w['Wi3'], r1(w['bi3']), w['Wf1'], r1(w['bf1']))
    xf = _decode(h, w['Wf2'], r1(w['bf2']))
    x = xf.reshape(NUM_NODES, NODE_DIM)

    pad16 = lambda m: jnp.pad(m, ((0, 13), (0, 0)))
    for l in range(N_BLOCKS):
        w1 = w['Wm1_%d' % l]
        x16 = jnp.pad(x, ((0, 0), (0, 13)))
        gd, gs = _sc_gather(x16, dstp, srcp)
        w1d16 = pad16(w1[0:3])
        w1s16 = pad16(w1[3:6])
        wbe = (eye8[None, :, :, None] * w1[6:10][:, None, None, :]).reshape(32, 512)
        m2q = _edge_mlp(gd, gs, eA,
                        jnp.kron(eye8, w1d16), jnp.kron(eye8, w1s16), wbe,
                        jnp.tile(w['bm1_%d' % l], 8).reshape(1, 512),
                        jnp.kron(eye8, w['Wm2_%d' % l]),
                        jnp.tile(w['bm2_%d' % l], 8).reshape(1, 512))
        aggr = _sc_scatter(dst2d, [q.reshape(E_PAD, 16) for q in m2q], zrows)
        wu1 = w['Wu1_%d' % l]
        x = _update(x, aggr,
                    wu1[0:3], [wu1[3 + 16 * q:3 + 16 * (q + 1)] for q in range(4)],
                    r1(w['bu1_%d' % l]),
                    w['Wu2_%d' % l], r1(w['bu2_%d' % l]))

    return x.reshape(1, NUM_NODES, NODE_DIM)


# speedup vs baseline: 4.5724x; 1.0542x over previous
import functools
import jax
import jax.numpy as jnp
from jax import lax
from jax.experimental import pallas as pl
from jax.experimental.pallas import tpu as pltpu
from jax.experimental.pallas import tpu_sc as plsc

NUM_NODES = 50000
NODE_DIM = 3
EDGE_DIM = 4
N_EDGES = 800000
N_BLOCKS = 2
FC_OUT = NUM_NODES * NODE_DIM

E_PAD = 819200

_DEC_TILE = 7680
_EDGE_TILE = 8192
_UPD_TILE = 5000

_G_PER_W = E_PAD // 32
_GSEG = 2560
_GCHUNK = 128

_S_ROWS_PER_W = (E_PAD // 128) // 16
_S_SEG_ROWS = 16
_S_SEG = _S_SEG_ROWS * 128
_TAB_ROWS_PER_W = NUM_NODES // 16


def _silu(x):
    return x * jax.nn.sigmoid(x)


def _head_body(p, wi1, bi1, wi2, bi2, wi3, bi3, wf1, bf1, h_out):
    z = _silu(jnp.dot(p[...], wi1[...], preferred_element_type=jnp.float32) + bi1[...])
    z = _silu(jnp.dot(z, wi2[...], preferred_element_type=jnp.float32) + bi2[...])
    z = jnp.dot(z, wi3[...], preferred_element_type=jnp.float32) + bi3[...]
    h_out[...] = _silu(jnp.dot(z, wf1[...], preferred_element_type=jnp.float32) + bf1[...])


def _head(p, wi1, bi1, wi2, bi2, wi3, bi3, wf1, bf1):
    return pl.pallas_call(
        _head_body,
        out_shape=jax.ShapeDtypeStruct((1, 256), jnp.float32),
    )(p, wi1, bi1, wi2, bi2, wi3, bi3, wf1, bf1)


def _dec_body(h, wf2, bf2, out):
    out[...] = jnp.dot(h[...], wf2[...], preferred_element_type=jnp.float32) + bf2[...]


def _decode(h, wf2, bf2):
    n_t = pl.cdiv(FC_OUT, _DEC_TILE)
    return pl.pallas_call(
        _dec_body,
        grid=(n_t,),
        in_specs=[
            pl.BlockSpec((1, 256), lambda j: (0, 0)),
            pl.BlockSpec((256, _DEC_TILE), lambda j: (0, j)),
            pl.BlockSpec((1, _DEC_TILE), lambda j: (0, j)),
        ],
        out_specs=pl.BlockSpec((1, _DEC_TILE), lambda j: (0, j)),
        out_shape=jax.ShapeDtypeStruct((1, FC_OUT), jnp.float32),
    )(h, wf2, bf2)


def _sc_gather(x4, dstp, srcp):
    mesh = plsc.VectorSubcoreMesh(core_axis_name="c", subcore_axis_name="s")

    @functools.partial(
        pl.kernel,
        mesh=mesh,
        out_type=[jax.ShapeDtypeStruct((E_PAD, 16), jnp.float32),
                  jax.ShapeDtypeStruct((E_PAD, 16), jnp.float32)],
        scratch_types=[pltpu.VMEM((_GSEG,), jnp.int32),
                       pltpu.VMEM((_GSEG, 16), jnp.float32),
                       pltpu.SemaphoreType.DMA],
        compiler_params=pltpu.CompilerParams(use_tc_tiling_on_sc=False),
    )
    def k(x_hbm, d_hbm, s_hbm, gd_hbm, gs_hbm, idx_v, rows_v, sem):
        wid = lax.axis_index("s") * 2 + lax.axis_index("c")
        base = wid * _G_PER_W

        def one(i_hbm, o_hbm):
            def seg(si, carry):
                off = pl.multiple_of(base + si * _GSEG, 8)
                pltpu.sync_copy(i_hbm.at[pl.ds(off, _GSEG)], idx_v)
                handles = []
                for ci in range(_GSEG // _GCHUNK):
                    co = ci * _GCHUNK
                    handles.append(pltpu.async_copy(
                        x_hbm.at[idx_v.at[pl.ds(co, _GCHUNK)]],
                        rows_v.at[pl.ds(co, _GCHUNK)], sem))
                for hd in handles:
                    hd.wait()
                pltpu.sync_copy(rows_v, o_hbm.at[pl.ds(off, _GSEG)])
                return carry

            lax.fori_loop(0, _G_PER_W // _GSEG, seg, 0)

        one(d_hbm, gd_hbm)
        one(s_hbm, gs_hbm)

    return k(x4, dstp, srcp)


def _sc_scatter(dst2d, m2q, zrows):
    mesh = plsc.VectorSubcoreMesh(core_axis_name="c", subcore_axis_name="s")

    @functools.partial(
        pl.kernel,
        mesh=mesh,
        out_type=[jax.ShapeDtypeStruct((NUM_NODES, 16), jnp.float32)] * 4,
        scratch_types=[pltpu.VMEM((_S_SEG_ROWS, 128), jnp.int32),
                       pltpu.VMEM((_S_SEG, 16), jnp.float32),
                       pltpu.VMEM_SHARED((NUM_NODES, 16), jnp.float32),
                       pltpu.SemaphoreType.DMA],
        compiler_params=pltpu.CompilerParams(use_tc_tiling_on_sc=False),
    )
    def k(d_hbm, m0, m1, m2, m3, z_hbm, o0, o1, o2, o3, idx_v, vals_v, table, sem):
        c = lax.axis_index("c")
        s = lax.axis_index("s")
        trow = pl.multiple_of(s * _TAB_ROWS_PER_W, 1)

        def one_pass(m_hbm, o_hbm):
            pltpu.sync_copy(z_hbm, table.at[pl.ds(trow, _TAB_ROWS_PER_W)])
            plsc.subcore_barrier()

            def seg(gi, carry):
                row0 = pl.multiple_of(s * _S_ROWS_PER_W + gi * _S_SEG_ROWS, 8)
                e0 = pl.multiple_of(row0 * 128, 8)
                pltpu.sync_copy(d_hbm.at[pl.ds(row0, _S_SEG_ROWS)], idx_v)
                pltpu.sync_copy(m_hbm.at[pl.ds(e0, _S_SEG)], vals_v)
                handles = []
                for j in range(_S_SEG_ROWS):
                    handles.append(pltpu.async_copy(
                        vals_v.at[pl.ds(j * 128, 128)],
                        table.at[idx_v.at[j]], sem, add=True))
                for hd in handles:
                    hd.wait()
                return carry

            lax.fori_loop(0, _S_ROWS_PER_W // _S_SEG_ROWS, seg, 0)
            plsc.subcore_barrier()
            pltpu.sync_copy(table.at[pl.ds(trow, _TAB_ROWS_PER_W)],
                            o_hbm.at[pl.ds(trow, _TAB_ROWS_PER_W)])

        @pl.when(c == 0)
        def _():
            one_pass(m0, o0)
            one_pass(m1, o1)

        @pl.when(c == 1)
        def _():
            one_pass(m2, o2)
            one_pass(m3, o3)

    return k(dst2d, *m2q, zrows)


def _edge_body(gd, gs, ea, wbd, wbs, wbe, b1p, w2bd, b2p, o0, o1, o2, o3):
    m = jnp.dot(gd[...], wbd[...], preferred_element_type=jnp.float32)
    m += jnp.dot(gs[...], wbs[...], preferred_element_type=jnp.float32)
    m += lax.dot_general(ea[...], wbe[...], (((0,), (0,)), ((), ())),
                         preferred_element_type=jnp.float32)
    m = _silu(m + b1p[...])
    m2 = _silu(jnp.dot(m, w2bd[...], preferred_element_type=jnp.float32) + b2p[...])
    row0 = pl.program_id(0) * (_EDGE_TILE // 8)
    rid = row0 + lax.broadcasted_iota(jnp.int32, (_EDGE_TILE // 8, 1), 0)
    m2 = jnp.where(rid < N_EDGES // 8, m2, 0.0)
    for q, o in enumerate((o0, o1, o2, o3)):
        o[...] = jnp.concatenate(
            [m2[:, 64 * j + 16 * q:64 * j + 16 * q + 16] for j in range(8)],
            axis=1)


def _edge_mlp(gd, gs, eA, wbd, wbs, wbe, b1p, w2bd, b2p):
    n_t = E_PAD // _EDGE_TILE
    rp = _EDGE_TILE // 8
    full = lambda r, c: pl.BlockSpec((r, c), lambda j: (0, 0))
    return pl.pallas_call(
        _edge_body,
        grid=(n_t,),
        in_specs=[
            pl.BlockSpec((rp, 128), lambda j: (j, 0)),
            pl.BlockSpec((rp, 128), lambda j: (j, 0)),
            pl.BlockSpec((32, rp), lambda j: (0, j)),
            full(128, 512), full(128, 512), full(32, 512), full(1, 512),
            full(512, 512), full(1, 512),
        ],
        out_specs=[pl.BlockSpec((rp, 128), lambda j: (j, 0))] * 4,
        out_shape=[jax.ShapeDtypeStruct((E_PAD // 8, 128), jnp.float32)] * 4,
    )(gd.reshape(E_PAD // 8, 128), gs.reshape(E_PAD // 8, 128),
      eA, wbd, wbs, wbe, b1p, w2bd, b2p)


def _upd_body(x, a0, a1, a2, a3, wux, wa0, wa1, wa2, wa3, b1, w2, b2, out):
    u = jnp.dot(x[...], wux[...], preferred_element_type=jnp.float32)
    u += jnp.dot(a0[...], wa0[...], preferred_element_type=jnp.float32)
    u += jnp.dot(a1[...], wa1[...], preferred_element_type=jnp.float32)
    u += jnp.dot(a2[...], wa2[...], preferred_element_type=jnp.float32)
    u += jnp.dot(a3[...], wa3[...], preferred_element_type=jnp.float32)
    u = _silu(u + b1[...])
    u = jnp.dot(u, w2[...], preferred_element_type=jnp.float32) + b2[...]
    out[...] = x[...] + u


def _update(x, aggr, wux, was, b1, w2, b2):
    n_t = NUM_NODES // _UPD_TILE
    full = lambda r, c: pl.BlockSpec((r, c), lambda j: (0, 0))
    return pl.pallas_call(
        _upd_body,
        grid=(n_t,),
        in_specs=[
            pl.BlockSpec((_UPD_TILE, 3), lambda j: (j, 0)),
            pl.BlockSpec((_UPD_TILE, 16), lambda j: (j, 0)),
            pl.BlockSpec((_UPD_TILE, 16), lambda j: (j, 0)),
            pl.BlockSpec((_UPD_TILE, 16), lambda j: (j, 0)),
            pl.BlockSpec((_UPD_TILE, 16), lambda j: (j, 0)),
            full(3, 64), full(16, 64), full(16, 64), full(16, 64),
            full(16, 64), full(1, 64), full(64, 3), full(1, 3),
        ],
        out_specs=pl.BlockSpec((_UPD_TILE, 3), lambda j: (j, 0)),
        out_shape=jax.ShapeDtypeStruct((NUM_NODES, 3), jnp.float32),
    )(x, *aggr, wux, *was, b1, w2, b2)


def kernel(params, edge_index, edge_attr, batch_ids, weights):
    w = weights
    r1 = lambda b: b.reshape(1, -1)
    pad_e = E_PAD - N_EDGES
    dstp = jnp.concatenate([edge_index[1], jnp.zeros((pad_e,), edge_index.dtype)])
    srcp = jnp.concatenate([edge_index[0], jnp.zeros((pad_e,), edge_index.dtype)])
    dst2d = dstp.reshape(E_PAD // 128, 128)
    eapT = jnp.pad(edge_attr.T, ((0, 0), (0, pad_e)))
    eA = eapT.reshape(4, E_PAD // 8, 8).transpose(0, 2, 1).reshape(32, E_PAD // 8)
    eye8 = jnp.eye(8, dtype=jnp.float32)
    zrows = jnp.zeros((_TAB_ROWS_PER_W, 16), jnp.float32)

    h = _head(params, w['Wi1'], r1(w['bi1']), w['Wi2'], r1(w['bi2']),
              w['Wi3'], r1(w['bi3']), w['Wf1'], r1(w['bf1']))
    xf = _decode(h, w['Wf2'], r1(w['bf2']))
    x = xf.reshape(NUM_NODES, NODE_DIM)

    pad16 = lambda m: jnp.pad(m, ((0, 13), (0, 0)))
    for l in range(N_BLOCKS):
        w1 = w['Wm1_%d' % l]
        x16 = jnp.pad(x, ((0, 0), (0, 13)))
        gd, gs = _sc_gather(x16, dstp, srcp)
        w1d16 = pad16(w1[0:3])
        w1s16 = pad16(w1[3:6])
        wbe = (eye8[None, :, :, None] * w1[6:10][:, None, None, :]).reshape(32, 512)
        m2q = _edge_mlp(gd, gs, eA,
                        jnp.kron(eye8, w1d16), jnp.kron(eye8, w1s16), wbe,
                        jnp.tile(w['bm1_%d' % l], 8).reshape(1, 512),
                        jnp.kron(eye8, w['Wm2_%d' % l]),
                        jnp.tile(w['bm2_%d' % l], 8).reshape(1, 512))
        aggr = _sc_scatter(dst2d, [q.reshape(E_PAD, 16) for q in m2q], zrows)
        wu1 = w['Wu1_%d' % l]
        x = _update(x, aggr,
                    wu1[0:3], [wu1[3 + 16 * q:3 + 16 * (q + 1)] for q in range(4)],
                    r1(w['bu1_%d' % l]),
                    w['Wu2_%d' % l], r1(w['bu2_%d' % l]))

    return x.reshape(1, NUM_NODES, NODE_DIM)
